# Initial kernel scaffold; baseline (speedup 1.0000x reference)
import jax
import jax.numpy as jnp
from jax.experimental import pallas as pl


def kernel(conv1_wT, conv1_b, conv2_wT, conv2_b, lin_cnn_w, lin_cnn_b, lin1_w, lin1_b, lin2_w, lin2_b, lstm_wx, lstm_ws, lstm_b, lin4_w, lin4_b, head_w, head_b, state, tensor_cv, sample_key):
    raise NotImplementedError("write your pallas kernel here")


# speedup vs baseline: 6.3608x; 6.3608x over previous
import math

import jax
import jax.numpy as jnp
from jax.experimental import pallas as pl
from jax.experimental.pallas import tpu as pltpu

_F32 = jnp.float32
_BF16 = jnp.bfloat16
_LOG2PI = math.log(2.0 * math.pi)

_W_Y = 16384
_W_V = 16000
_W_H = 15997
_W_C2 = 15106
_W_Q = 14848


def _cnn_state_kernel(ph_ref, wq_ref, b1_ref, w2t_ref, b2_ref, w3_ref,
                      st_ref, w1_ref, oxp_ref, osp_ref, xs_ref, scq_ref):
    ph = ph_ref[...]
    y = (jnp.dot(wq_ref[0], ph[:, 0:_W_Y], preferred_element_type=_F32)
         + jnp.dot(wq_ref[1], ph[:, 1:1 + _W_Y], preferred_element_type=_F32)
         + jnp.dot(wq_ref[2], ph[:, 128:128 + _W_Y], preferred_element_type=_F32)
         + jnp.dot(wq_ref[3], ph[:, 129:129 + _W_Y], preferred_element_type=_F32))
    c1 = jnp.maximum(y + b1_ref[...], 0.0)

    v = jnp.maximum(jnp.maximum(c1[:, 0:_W_V], c1[:, 128:128 + _W_V]),
                    jnp.maximum(c1[:, 256:256 + _W_V], c1[:, 384:384 + _W_V]))
    hm = jnp.maximum(jnp.maximum(v[:, 0:_W_H], v[:, 1:1 + _W_H]),
                     jnp.maximum(v[:, 2:2 + _W_H], v[:, 3:3 + _W_H]))
    p1 = hm.astype(_BF16)

    a2 = None
    for t in range(16):
        dy, dx = divmod(t, 4)
        off = dy * 256 + 2 * dx
        d = jnp.dot(w2t_ref[0, t], p1[:, off:off + _W_C2],
                    preferred_element_type=_F32)
        a2 = d if a2 is None else a2 + d
    c2 = jnp.maximum(a2 + b2_ref[0], 0.0)

    q = jnp.maximum(jnp.maximum(c2[:, 0:_W_Q], c2[:, 2:2 + _W_Q]),
                    jnp.maximum(c2[:, 256:256 + _W_Q], c2[:, 258:258 + _W_Q]))
    qb = q.astype(_BF16)

    for i in range(29):
        scq_ref[8 * i:8 * i + 8, :] = qb[:, 512 * i:512 * i + 512]
    li = jax.lax.broadcasted_iota(jnp.int32, (512, 32), 0)
    ci = jax.lax.broadcasted_iota(jnp.int32, (512, 32), 1)
    sel = jnp.where((li == 4 * ci) & (ci < 29), 1.0, 0.0).astype(_BF16)
    dm = jnp.dot(scq_ref[...], sel, preferred_element_type=_F32)
    db = dm.astype(_BF16)

    xs_ref[...] = jnp.zeros((8, 896), _BF16)
    for i in range(29):
        xs_ref[:, 29 * i:29 * i + 29] = db[8 * i:8 * i + 8, 0:29]
    x8 = xs_ref[...]

    part = None
    for co in range(8):
        d = jnp.dot(x8[co:co + 1, :], w3_ref[0, co], preferred_element_type=_F32)
        part = d if part is None else part + d
    oxp_ref[...] = part.reshape(1, 1, 256)

    sp = jnp.dot(st_ref[...], w1_ref[...], preferred_element_type=_F32)
    osp_ref[...] = sp.reshape(1, 1, 32)


def _tail_kernel(xp_ref, sp_ref, b3_ref, b1_ref, w2l_ref, b2l_ref,
                 wlx_ref, wls_ref, bl_ref, w4_ref, b4_ref, wh_ref, bh_ref,
                 eps_ref, oa_ref, olp_ref, oe_ref):
    a = wh_ref.shape[1] // 2
    x_cnn = jnp.maximum(xp_ref[0] + xp_ref[1] + b3_ref[...], 0.0)
    o1 = jnp.maximum(sp_ref[0] + sp_ref[1] + b1_ref[...], 0.0)
    o2 = jnp.maximum(jnp.dot(o1, w2l_ref[...], preferred_element_type=_F32)
                     + b2l_ref[...], 0.0)
    gates = (jnp.dot(x_cnn, wlx_ref[...], preferred_element_type=_F32)
             + jnp.dot(o2, wls_ref[...], preferred_element_type=_F32)
             + bl_ref[...])
    i_g = 1.0 / (1.0 + jnp.exp(-gates[:, 0:64]))
    g_g = jnp.tanh(gates[:, 64:128])
    o_g = 1.0 / (1.0 + jnp.exp(-gates[:, 128:192]))
    h = o_g * jnp.tanh(i_g * g_g)
    o4 = jnp.maximum(jnp.dot(h, w4_ref[...], preferred_element_type=_F32)
                     + b4_ref[...], 0.0)
    heads = jnp.dot(o4, wh_ref[...], preferred_element_type=_F32) + bh_ref[...]
    mu = jnp.tanh(heads[:, 0:a])
    sig = jnp.maximum(heads[:, a:], 0.0) + 0.001

    hld = 0.5 * jnp.sum(jnp.log(sig), axis=1, keepdims=True)
    oe_ref[...] = 0.5 * a * (1.0 + _LOG2PI) + hld
    ri = jax.lax.broadcasted_iota(jnp.int32, (a, a), 0)
    ci = jax.lax.broadcasted_iota(jnp.int32, (a, a), 1)
    loc = jnp.where(ri == ci, mu, 0.0)
    act = loc + jnp.sqrt(sig) * eps_ref[...]
    oa_ref[...] = act
    diff = act - loc
    olp_ref[...] = (-0.5 * jnp.sum(diff * diff / sig, axis=1, keepdims=True)
                    - hld - 0.5 * a * _LOG2PI)


def kernel(conv1_wT, conv1_b, conv2_wT, conv2_b, lin_cnn_w, lin_cnn_b,
           lin1_w, lin1_b, lin2_w, lin2_b, lstm_wx, lstm_ws, lstm_b,
           lin4_w, lin4_b, head_w, head_b, state, tensor_cv, sample_key):
    a = head_w.shape[1] // 2
    s = state.shape[1]

    ph = tensor_cv.astype(_BF16).reshape(3, 125, 4, 125, 4)
    ph = ph.transpose(2, 4, 0, 1, 3)
    ph = jnp.pad(ph, ((0, 0), (0, 0), (0, 0), (0, 5), (0, 3)))
    ph = ph.reshape(48, 130 * 128)

    wq = conv1_wT.reshape(8, 2, 4, 2, 4, 3).transpose(1, 3, 0, 2, 4, 5)
    wq = wq.reshape(4, 8, 48)

    w2t = conv2_wT.reshape(2, 8, 4, 4, 8).transpose(0, 2, 3, 1, 4)
    w2t = w2t.reshape(2, 16, 8, 8)

    xp, sp = pl.pallas_call(
        _cnn_state_kernel,
        out_shape=[jax.ShapeDtypeStruct((2, 1, 256), _F32),
                   jax.ShapeDtypeStruct((2, 1, 32), _F32)],
        grid=(2,),
        in_specs=[
            pl.BlockSpec((48, 130 * 128), lambda g: (0, 0)),
            pl.BlockSpec((4, 8, 48), lambda g: (0, 0, 0)),
            pl.BlockSpec((8, 1), lambda g: (0, 0)),
            pl.BlockSpec((1, 16, 8, 8), lambda g: (g, 0, 0, 0)),
            pl.BlockSpec((1, 8, 1), lambda g: (g, 0, 0)),
            pl.BlockSpec((1, 8, 896, 256), lambda g: (g, 0, 0, 0)),
            pl.BlockSpec((1, s // 2), lambda g: (0, g)),
            pl.BlockSpec((s // 2, 32), lambda g: (g, 0)),
        ],
        out_specs=[pl.BlockSpec((1, 1, 256), lambda g: (g, 0, 0)),
                   pl.BlockSpec((1, 1, 32), lambda g: (g, 0, 0))],
        scratch_shapes=[pltpu.VMEM((8, 896), _BF16),
                        pltpu.VMEM((232, 512), _BF16)],
        compiler_params=pltpu.CompilerParams(dimension_semantics=("parallel",)),
    )(ph, wq, conv1_b, w2t, conv2_b, lin_cnn_w, state, lin1_w)

    eps = jax.random.normal(jax.random.wrap_key_data(sample_key),
                            (1, a, a), _F32).reshape(a, a)

    act, lp, ent = pl.pallas_call(
        _tail_kernel,
        out_shape=[jax.ShapeDtypeStruct((a, a), _F32),
                   jax.ShapeDtypeStruct((a, 1), _F32),
                   jax.ShapeDtypeStruct((1, 1), _F32)],
    )(xp, sp, lin_cnn_b, lin1_b, lin2_w, lin2_b, lstm_wx, lstm_ws, lstm_b,
      lin4_w, lin4_b, head_w, head_b, eps)

    return act.reshape(1, a, a), lp.reshape(1, a), ent.reshape(())


# speedup vs baseline: 14.7165x; 2.3136x over previous
import math

import jax
import jax.numpy as jnp
from jax.experimental import pallas as pl
from jax.experimental.pallas import tpu as pltpu

_F32 = jnp.float32
_BF16 = jnp.bfloat16
_LOG2PI = math.log(2.0 * math.pi)

_W_RP = 65024
_W_Y = 64000
_W_V = 62464
_W_H = 62452
_W_C2 = 58888
_W_Q = 57856


def _cnn_state_kernel(rp_ref, wq_ref, b1_ref, w2t_ref, b2_ref, w3_ref,
                      st_ref, w1_ref, oxp_ref, osp_ref,
                      xs_ref, scq_ref, acc1_ref, acc2_ref, p1_ref, rpr_ref):
    acc1_ref[...] = jnp.zeros(acc1_ref.shape, _F32)
    rpr_ref[...] = rp_ref[:, 0:_W_Y + 512]

    def c1_body(dx, _):
        sdx = rpr_ref[...]
        acc1_ref[...] = (acc1_ref[...]
                         + jnp.dot(wq_ref[dx], sdx[:, 0:_W_Y],
                                   preferred_element_type=_F32)
                         + jnp.dot(wq_ref[8 + dx], sdx[:, 512:512 + _W_Y],
                                   preferred_element_type=_F32))
        rpr_ref[...] = pltpu.roll(sdx, _W_Y + 512 - 1, axis=1)
        return 0

    jax.lax.fori_loop(0, 8, c1_body, 0)
    c1 = jnp.maximum(acc1_ref[...] + b1_ref[...], 0.0).astype(_BF16)

    v = jnp.maximum(jnp.maximum(c1[:, 0:_W_V], c1[:, 512:512 + _W_V]),
                    jnp.maximum(c1[:, 1024:1024 + _W_V],
                                c1[:, 1536:1536 + _W_V]))
    p1_ref[...] = jnp.maximum(jnp.maximum(v[:, 0:_W_H], v[:, 4:4 + _W_H]),
                              jnp.maximum(v[:, 8:8 + _W_H],
                                          v[:, 12:12 + _W_H]))

    acc2_ref[...] = jnp.zeros(acc2_ref.shape, _F32)

    def c2_body(dx, _):
        base = p1_ref[...]
        a = acc2_ref[...]
        for dy in range(4):
            a = a + jnp.dot(w2t_ref[0, 4 * dy + dx],
                            base[:, 1024 * dy:1024 * dy + _W_C2],
                            preferred_element_type=_F32)
        acc2_ref[...] = a
        p1_ref[...] = pltpu.roll(base, _W_H - 8, axis=1)
        return 0

    jax.lax.fori_loop(0, 4, c2_body, 0)
    c2 = jnp.maximum(acc2_ref[...] + b2_ref[0], 0.0).astype(_BF16)

    qb = jnp.maximum(jnp.maximum(c2[:, 0:_W_Q], c2[:, 8:8 + _W_Q]),
                     jnp.maximum(c2[:, 1024:1024 + _W_Q],
                                 c2[:, 1032:1032 + _W_Q]))

    scq_ref[...] = jnp.zeros(scq_ref.shape, _BF16)
    for i in range(28):
        scq_ref[8 * i:8 * i + 8, :] = qb[:, 2048 * i:2048 * i + 2048]
    scq_ref[224:232, 0:512] = qb[:, 2048 * 28:2048 * 28 + 512]
    li = jax.lax.broadcasted_iota(jnp.int32, (2048, 32), 0)
    ci = jax.lax.broadcasted_iota(jnp.int32, (2048, 32), 1)
    sel = jnp.where((li == 16 * ci) & (ci < 29), 1.0, 0.0).astype(_BF16)
    dm = jnp.dot(scq_ref[...], sel, preferred_element_type=_F32)
    db = dm.astype(_BF16)

    xs_ref[...] = jnp.zeros((8, 896), _BF16)
    for i in range(29):
        xs_ref[:, 29 * i:29 * i + 29] = db[8 * i:8 * i + 8, 0:29]
    x8 = xs_ref[...]

    part = None
    for co in range(8):
        d = jnp.dot(x8[co:co + 1, :], w3_ref[0, co], preferred_element_type=_F32)
        part = d if part is None else part + d
    oxp_ref[...] = part.reshape(1, 1, 256)

    sp = jnp.dot(st_ref[...], w1_ref[...], preferred_element_type=_F32)
    osp_ref[...] = sp.reshape(1, 1, 32)


def _tail_kernel(xp_ref, sp_ref, b3_ref, b1_ref, w2l_ref, b2l_ref,
                 wlx_ref, wls_ref, bl_ref, w4_ref, b4_ref, wh_ref, bh_ref,
                 eps_ref, oa_ref, olp_ref, oe_ref):
    a = wh_ref.shape[1] // 2
    x_cnn = jnp.maximum(xp_ref[0] + xp_ref[1] + b3_ref[...], 0.0)
    o1 = jnp.maximum(sp_ref[0] + sp_ref[1] + b1_ref[...], 0.0)
    o2 = jnp.maximum(jnp.dot(o1, w2l_ref[...], preferred_element_type=_F32)
                     + b2l_ref[...], 0.0)
    gates = (jnp.dot(x_cnn, wlx_ref[...], preferred_element_type=_F32)
             + jnp.dot(o2, wls_ref[...], preferred_element_type=_F32)
             + bl_ref[...])
    i_g = 1.0 / (1.0 + jnp.exp(-gates[:, 0:64]))
    g_g = jnp.tanh(gates[:, 64:128])
    o_g = 1.0 / (1.0 + jnp.exp(-gates[:, 128:192]))
    h = o_g * jnp.tanh(i_g * g_g)
    o4 = jnp.maximum(jnp.dot(h, w4_ref[...], preferred_element_type=_F32)
                     + b4_ref[...], 0.0)
    heads = jnp.dot(o4, wh_ref[...], preferred_element_type=_F32) + bh_ref[...]
    mu = jnp.tanh(heads[:, 0:a])
    sig = jnp.maximum(heads[:, a:], 0.0) + 0.001

    hld = 0.5 * jnp.sum(jnp.log(sig), axis=1, keepdims=True)
    oe_ref[...] = 0.5 * a * (1.0 + _LOG2PI) + hld
    ri = jax.lax.broadcasted_iota(jnp.int32, (a, a), 0)
    ci = jax.lax.broadcasted_iota(jnp.int32, (a, a), 1)
    loc = jnp.where(ri == ci, mu, 0.0)
    act = loc + jnp.sqrt(sig) * eps_ref[...]
    oa_ref[...] = act
    diff = act - loc
    olp_ref[...] = (-0.5 * jnp.sum(diff * diff / sig, axis=1, keepdims=True)
                    - hld - 0.5 * a * _LOG2PI)


def kernel(conv1_wT, conv1_b, conv2_wT, conv2_b, lin_cnn_w, lin_cnn_b,
           lin1_w, lin1_b, lin2_w, lin2_b, lstm_wx, lstm_ws, lstm_b,
           lin4_w, lin4_b, head_w, head_b, state, tensor_cv, sample_key):
    a = head_w.shape[1] // 2
    s = state.shape[1]

    rp = tensor_cv.reshape(3, 125, 4, 500).transpose(2, 0, 1, 3).astype(_BF16)
    rp = jnp.pad(rp, ((0, 0), (0, 0), (0, 2), (0, 12)))
    rp = rp.reshape(12, _W_RP)

    wq = conv1_wT.reshape(8, 2, 4, 8, 3).transpose(1, 3, 0, 2, 4)
    wq = wq.reshape(16, 8, 12)

    w2t = conv2_wT.reshape(2, 8, 4, 4, 8).transpose(0, 2, 3, 1, 4)
    w2t = w2t.reshape(2, 16, 8, 8)

    xp, sp = pl.pallas_call(
        _cnn_state_kernel,
        out_shape=[jax.ShapeDtypeStruct((2, 1, 256), _F32),
                   jax.ShapeDtypeStruct((2, 1, 32), _F32)],
        grid=(2,),
        in_specs=[
            pl.BlockSpec((12, _W_RP), lambda g: (0, 0)),
            pl.BlockSpec((16, 8, 12), lambda g: (0, 0, 0)),
            pl.BlockSpec((8, 1), lambda g: (0, 0)),
            pl.BlockSpec((1, 16, 8, 8), lambda g: (g, 0, 0, 0)),
            pl.BlockSpec((1, 8, 1), lambda g: (g, 0, 0)),
            pl.BlockSpec((1, 8, 896, 256), lambda g: (g, 0, 0, 0)),
            pl.BlockSpec((1, s // 2), lambda g: (0, g)),
            pl.BlockSpec((s // 2, 32), lambda g: (g, 0)),
        ],
        out_specs=[pl.BlockSpec((1, 1, 256), lambda g: (g, 0, 0)),
                   pl.BlockSpec((1, 1, 32), lambda g: (g, 0, 0))],
        scratch_shapes=[pltpu.VMEM((8, 896), _BF16),
                        pltpu.VMEM((232, 2048), _BF16),
                        pltpu.VMEM((8, _W_Y), _F32),
                        pltpu.VMEM((8, _W_C2), _F32),
                        pltpu.VMEM((8, _W_H), _BF16),
                        pltpu.VMEM((12, _W_Y + 512), _BF16)],
        compiler_params=pltpu.CompilerParams(dimension_semantics=("parallel",)),
    )(rp, wq, conv1_b, w2t, conv2_b, lin_cnn_w, state, lin1_w)

    eps = jax.random.normal(jax.random.wrap_key_data(sample_key),
                            (1, a, a), _F32).reshape(a, a)

    act, lp, ent = pl.pallas_call(
        _tail_kernel,
        out_shape=[jax.ShapeDtypeStruct((a, a), _F32),
                   jax.ShapeDtypeStruct((a, 1), _F32),
                   jax.ShapeDtypeStruct((1, 1), _F32)],
    )(xp, sp, lin_cnn_b, lin1_b, lin2_w, lin2_b, lstm_wx, lstm_ws, lstm_b,
      lin4_w, lin4_b, head_w, head_b, eps)

    return act.reshape(1, a, a), lp.reshape(1, a), ent.reshape(())
